# baseline (speedup 1.0000x reference)
import functools

import jax
import jax.numpy as jnp
from jax import lax
from jax.experimental import pallas as pl
from jax.experimental.pallas import tpu as pltpu
from jax.experimental.pallas import tpu_sc as plsc

L = 16
NC, NS = 2, 16
NW = NC * NS
NBUF = 8


@functools.lru_cache(maxsize=None)
def _build_bias(V, B):
    bpw = B // NW
    chunk = min(bpw, 128)
    nch = bpw // chunk
    mesh = plsc.VectorSubcoreMesh(core_axis_name="c", subcore_axis_name="s")

    @functools.partial(
        pl.kernel,
        mesh=mesh,
        out_type=jax.ShapeDtypeStruct((B,), jnp.float32),
        compiler_params=pltpu.CompilerParams(
            needs_layout_passes=False, use_tc_tiling_on_sc=False),
        scratch_types=[
            pltpu.VMEM((nch, chunk), jnp.int32),
            pltpu.VMEM((nch, chunk), jnp.int32),
            pltpu.VMEM((bpw, 1), jnp.float32),
            pltpu.VMEM((bpw, 1), jnp.float32),
            pltpu.VMEM((bpw,), jnp.float32),
            pltpu.SemaphoreType.DMA,
        ],
    )
    def bias_sum(ii_hbm, jj_hbm, bi_hbm, bj_hbm, out_hbm,
                 ii_v, jj_v, bi_v, bj_v, out_v, sem):
        wid = lax.axis_index("s") * NC + lax.axis_index("c")
        base = wid * bpw
        pltpu.sync_copy(ii_hbm.at[pl.ds(wid * nch, nch)], ii_v)
        pltpu.sync_copy(jj_hbm.at[pl.ds(wid * nch, nch)], jj_v)
        copies = []
        for c in range(nch):
            s = pl.ds(c * chunk, chunk)
            copies.append(pltpu.async_copy(bi_hbm.at[ii_v.at[c]], bi_v.at[s, :], sem))
            copies.append(pltpu.async_copy(bj_hbm.at[jj_v.at[c]], bj_v.at[s, :], sem))
        for cp in copies:
            cp.wait()

        lane = lax.iota(jnp.int32, L)

        def block(b0, _):
            rows = b0 * L + lane
            zcol = jnp.zeros((L,), jnp.int32)
            out_v[pl.ds(b0 * L, L)] = (plsc.load_gather(bi_v, [rows, zcol])
                                       + plsc.load_gather(bj_v, [rows, zcol]))
            return 0

        lax.fori_loop(0, bpw // L, block, 0)
        pltpu.sync_copy(out_v, out_hbm.at[pl.ds(base, bpw)])

    return bias_sum


@functools.lru_cache(maxsize=None)
def _build_dot(V, D, B):
    bpw = B // NW
    chunk = min(bpw, 128)
    nch = bpw // chunk
    vfull = V - V % 128
    nft = D // 8
    mesh = plsc.VectorSubcoreMesh(core_axis_name="c", subcore_axis_name="s")

    @functools.partial(
        pl.kernel,
        mesh=mesh,
        out_type=jax.ShapeDtypeStruct((B,), jnp.float32),
        compiler_params=pltpu.CompilerParams(
            needs_layout_passes=False, use_tc_tiling_on_sc=True),
        scratch_types=[
            pltpu.VMEM((bpw,), jnp.int32),
            pltpu.VMEM((bpw,), jnp.int32),
            pltpu.VMEM((bpw,), jnp.float32),
            pltpu.VMEM((NBUF, nft, 8, 128), jnp.float32),
            pltpu.VMEM((NBUF, nft, 8, 128), jnp.float32),
            pltpu.VMEM((nft, 8, 128), jnp.float32),
            pltpu.VMEM((nft, 8, 128), jnp.float32),
            pltpu.VMEM((bpw,), jnp.float32),
        ] + [pltpu.SemaphoreType.DMA] * (NBUF + 1),
    )
    def dot(ii_hbm, jj_hbm, bs_hbm, wit_hbm, wjt_hbm, wtl_hbm, wtr_hbm,
            out_hbm,
            ii_v, jj_v, bs_v, ring_i, ring_j, tail_i, tail_j,
            out_v, *sems):
        wid = lax.axis_index("s") * NC + lax.axis_index("c")
        base = wid * bpw
        tsem = sems[NBUF]
        pltpu.sync_copy(ii_hbm.at[pl.ds(base, bpw)], ii_v)
        pltpu.sync_copy(jj_hbm.at[pl.ds(base, bpw)], jj_v)

        head = [pltpu.async_copy(wtl_hbm, tail_i, tsem),
                pltpu.async_copy(wtr_hbm, tail_j, tsem),
                pltpu.async_copy(bs_hbm.at[pl.ds(base, bpw)], bs_v, tsem)]

        lane = lax.iota(jnp.int32, L)
        ft_lo, s_lo = lane // 8, lane % 8
        ft_hi, s_hi = ft_lo + 2, s_lo
        max_tile = jnp.int32(vfull // 128 - 1)

        def fire(slot, vi, vj):
            ti = pl.multiple_of(jnp.minimum(vi >> 7, max_tile) << 7, 128)
            tj = pl.multiple_of(jnp.minimum(vj >> 7, max_tile) << 7, 128)
            pltpu.async_copy(wit_hbm.at[:, :, pl.ds(ti, 128)],
                             ring_i.at[slot], sems[slot])
            pltpu.async_copy(wjt_hbm.at[:, :, pl.ds(tj, 128)],
                             ring_j.at[slot], sems[slot])

        def drain(slot):
            pltpu.make_async_copy(
                wit_hbm.at[:, :, pl.ds(0, 128)], ring_i.at[slot],
                sems[slot]).wait()
            pltpu.make_async_copy(
                wit_hbm.at[:, :, pl.ds(0, 128)], ring_j.at[slot],
                sems[slot]).wait()

        def gather32(src, col):
            return (plsc.load_gather(src, [ft_lo, s_lo, col]),
                    plsc.load_gather(src, [ft_hi, s_hi, col]))

        def retire(slot, vi, vj, k, bsum):
            gi0, gi1 = gather32(ring_i.at[slot], jnp.broadcast_to(vi & 127, (L,)))
            gj0, gj1 = gather32(ring_j.at[slot], jnp.broadcast_to(vj & 127, (L,)))
            s = jnp.sum(gi0 * gj0 + gi1 * gj1) + bsum
            kk = jnp.broadcast_to(k, (L,))
            plsc.store_scatter(out_v, [kk], jnp.broadcast_to(s, (L,)),
                               mask=lane == 0)

            @pl.when(jnp.logical_or(vi >= vfull, vj >= vfull))
            def _():
                ci = jnp.broadcast_to(jnp.clip(vi - (V - 128), 0, 127), (L,))
                cj = jnp.broadcast_to(jnp.clip(vj - (V - 128), 0, 127), (L,))
                ti0, ti1 = gather32(tail_i, ci)
                tj0, tj1 = gather32(tail_j, cj)
                ri0, ri1 = gather32(ring_i.at[slot],
                                    jnp.broadcast_to(vi & 127, (L,)))
                rj0, rj1 = gather32(ring_j.at[slot],
                                    jnp.broadcast_to(vj & 127, (L,)))
                xi0 = jnp.where(vi >= vfull, ti0, ri0)
                xi1 = jnp.where(vi >= vfull, ti1, ri1)
                xj0 = jnp.where(vj >= vfull, tj0, rj0)
                xj1 = jnp.where(vj >= vfull, tj1, rj1)
                st = jnp.sum(xi0 * xj0 + xi1 * xj1) + bsum
                plsc.store_scatter(out_v, [kk], jnp.broadcast_to(st, (L,)),
                                   mask=lane == 0)

        iv0 = ii_v[pl.ds(0, L)]
        jv0 = jj_v[pl.ds(0, L)]
        for r in range(NBUF):
            fire(r, iv0[r], jv0[r])
        for cp in head:
            cp.wait()

        def group(g, carry):
            iv_p, jv_p, bs_p = carry
            iv = ii_v[pl.ds(g * L, L)]
            jv = jj_v[pl.ds(g * L, L)]
            bs = bs_v[pl.ds(g * L, L)]
            for r in range(L):
                slot = r % NBUF
                k = g * L + r
                if r < NBUF:
                    @pl.when(g > 0)
                    def _(slot=slot, rp=r + L - NBUF, iv_p=iv_p, jv_p=jv_p,
                          bs_p=bs_p, kp=k - NBUF):
                        drain(slot)
                        retire(slot, iv_p[rp], jv_p[rp], kp, bs_p[rp])
                    @pl.when(g > 0)
                    def _(slot=slot, r=r, iv=iv, jv=jv):
                        fire(slot, iv[r], jv[r])
                else:
                    drain(slot)
                    rp = r - NBUF
                    retire(slot, iv[rp], jv[rp], k - NBUF, bs[rp])
                    fire(slot, iv[r], jv[r])
            return iv, jv, bs

        ng = bpw // L
        lax.fori_loop(0, ng, group, (iv0, jv0, jnp.zeros((L,), jnp.float32)))

        ivl = ii_v[pl.ds(bpw - L, L)]
        jvl = jj_v[pl.ds(bpw - L, L)]
        bsl = bs_v[pl.ds(bpw - L, L)]
        for r in range(L - NBUF, L):
            slot = r % NBUF
            drain(slot)
            retire(slot, ivl[r], jvl[r], (ng - 1) * L + r, bsl[r])

        pltpu.sync_copy(out_v, out_hbm.at[pl.ds(base, bpw)])

    return dot


def kernel(i_indices, j_indices, wi, wj, bi, bj):
    V, D = wi.shape
    B = i_indices.shape[0]
    bpw = B // NW
    chunk = min(bpw, 128)
    ii = i_indices.astype(jnp.int32)
    jj = j_indices.astype(jnp.int32)
    ii2 = ii.reshape(B // chunk, chunk)
    jj2 = jj.reshape(B // chunk, chunk)
    bsum = _build_bias(V, B)(ii2, jj2, bi, bj)
    wit = wi.T.reshape(D // 8, 8, V)
    wjt = wj.T.reshape(D // 8, 8, V)
    return _build_dot(V, D, B)(ii, jj, bsum, wit, wjt,
                               wit[:, :, V - 128:], wjt[:, :, V - 128:])


# speedup vs baseline: 6.5709x; 6.5709x over previous
import functools

import jax
import jax.numpy as jnp
from jax import lax
from jax.experimental import pallas as pl
from jax.experimental.pallas import tpu as pltpu
from jax.experimental.pallas import tpu_sc as plsc

L = 16
NC, NS = 2, 16
NW = NC * NS
NBUF = 8


@functools.lru_cache(maxsize=None)
def _build_dot(V, D, B):
    bpw = B // NW
    chunk = min(bpw, 128)
    nch = bpw // chunk
    vfull = V - V % 128
    nft = D // 8
    mesh = plsc.VectorSubcoreMesh(core_axis_name="c", subcore_axis_name="s")

    @functools.partial(
        pl.kernel,
        mesh=mesh,
        out_type=jax.ShapeDtypeStruct((B,), jnp.float32),
        compiler_params=pltpu.CompilerParams(
            needs_layout_passes=False, use_tc_tiling_on_sc=True),
        scratch_types=[
            pltpu.VMEM((bpw,), jnp.int32),
            pltpu.VMEM((bpw,), jnp.int32),
            pltpu.VMEM((bpw,), jnp.float32),
            pltpu.VMEM((bpw,), jnp.float32),
            pltpu.VMEM((NBUF, nft, 8, 128), jnp.float32),
            pltpu.VMEM((NBUF, nft, 8, 128), jnp.float32),
            pltpu.VMEM((nft, 8, 128), jnp.float32),
            pltpu.VMEM((nft, 8, 128), jnp.float32),
            pltpu.VMEM((bpw,), jnp.float32),
        ] + [pltpu.SemaphoreType.DMA] * (NBUF + 1),
    )
    def dot(ii_hbm, jj_hbm, wit_hbm, wjt_hbm, wtl_hbm, wtr_hbm,
            bi_hbm, bj_hbm, out_hbm,
            ii_v, jj_v, bi_v, bj_v, ring_i, ring_j, tail_i, tail_j,
            out_v, *sems):
        wid = lax.axis_index("s") * NC + lax.axis_index("c")
        base = wid * bpw
        tsem = sems[NBUF]
        pltpu.sync_copy(ii_hbm.at[pl.ds(base, bpw)], ii_v)
        pltpu.sync_copy(jj_hbm.at[pl.ds(base, bpw)], jj_v)

        head = [pltpu.async_copy(wtl_hbm, tail_i, tsem),
                pltpu.async_copy(wtr_hbm, tail_j, tsem)]
        for c in range(nch):
            s = pl.ds(c * chunk, chunk)
            head.append(pltpu.async_copy(bi_hbm.at[ii_v.at[s]], bi_v.at[s], tsem))
            head.append(pltpu.async_copy(bj_hbm.at[jj_v.at[s]], bj_v.at[s], tsem))

        lane = lax.iota(jnp.int32, L)
        ft_lo, s_lo = lane // 8, lane % 8
        ft_hi, s_hi = ft_lo + 2, s_lo
        max_tile = jnp.int32(vfull // 128 - 1)

        def fire(slot, vi, vj):
            ti = pl.multiple_of(jnp.minimum(vi >> 7, max_tile) << 7, 128)
            tj = pl.multiple_of(jnp.minimum(vj >> 7, max_tile) << 7, 128)
            pltpu.async_copy(wit_hbm.at[:, :, pl.ds(ti, 128)],
                             ring_i.at[slot], sems[slot])
            pltpu.async_copy(wjt_hbm.at[:, :, pl.ds(tj, 128)],
                             ring_j.at[slot], sems[slot])

        def drain(slot):
            pltpu.make_async_copy(
                wit_hbm.at[:, :, pl.ds(0, 128)], ring_i.at[slot],
                sems[slot]).wait()
            pltpu.make_async_copy(
                wit_hbm.at[:, :, pl.ds(0, 128)], ring_j.at[slot],
                sems[slot]).wait()

        def gather32(src, col):
            return (plsc.load_gather(src, [ft_lo, s_lo, col]),
                    plsc.load_gather(src, [ft_hi, s_hi, col]))

        def retire(slot, vi, vj, k, bsum):
            gi0, gi1 = gather32(ring_i.at[slot], jnp.broadcast_to(vi & 127, (L,)))
            gj0, gj1 = gather32(ring_j.at[slot], jnp.broadcast_to(vj & 127, (L,)))
            s = jnp.sum(gi0 * gj0 + gi1 * gj1) + bsum
            kk = jnp.broadcast_to(k, (L,))
            plsc.store_scatter(out_v, [kk], jnp.broadcast_to(s, (L,)),
                               mask=lane == 0)

            @pl.when(jnp.logical_or(vi >= vfull, vj >= vfull))
            def _():
                ci = jnp.broadcast_to(jnp.clip(vi - (V - 128), 0, 127), (L,))
                cj = jnp.broadcast_to(jnp.clip(vj - (V - 128), 0, 127), (L,))
                ti0, ti1 = gather32(tail_i, ci)
                tj0, tj1 = gather32(tail_j, cj)
                ri0, ri1 = gather32(ring_i.at[slot],
                                    jnp.broadcast_to(vi & 127, (L,)))
                rj0, rj1 = gather32(ring_j.at[slot],
                                    jnp.broadcast_to(vj & 127, (L,)))
                xi0 = jnp.where(vi >= vfull, ti0, ri0)
                xi1 = jnp.where(vi >= vfull, ti1, ri1)
                xj0 = jnp.where(vj >= vfull, tj0, rj0)
                xj1 = jnp.where(vj >= vfull, tj1, rj1)
                st = jnp.sum(xi0 * xj0 + xi1 * xj1) + bsum
                plsc.store_scatter(out_v, [kk], jnp.broadcast_to(st, (L,)),
                                   mask=lane == 0)

        iv0 = ii_v[pl.ds(0, L)]
        jv0 = jj_v[pl.ds(0, L)]
        for r in range(NBUF):
            fire(r, iv0[r], jv0[r])
        for cp in head:
            cp.wait()

        def group(g, carry):
            iv_p, jv_p, bs_p = carry
            iv = ii_v[pl.ds(g * L, L)]
            jv = jj_v[pl.ds(g * L, L)]
            bs = bi_v[pl.ds(g * L, L)] + bj_v[pl.ds(g * L, L)]
            for r in range(L):
                slot = r % NBUF
                k = g * L + r
                if r < NBUF:
                    @pl.when(g > 0)
                    def _(slot=slot, rp=r + L - NBUF, iv_p=iv_p, jv_p=jv_p,
                          bs_p=bs_p, kp=k - NBUF):
                        drain(slot)
                        retire(slot, iv_p[rp], jv_p[rp], kp, bs_p[rp])
                    @pl.when(g > 0)
                    def _(slot=slot, r=r, iv=iv, jv=jv):
                        fire(slot, iv[r], jv[r])
                else:
                    drain(slot)
                    rp = r - NBUF
                    retire(slot, iv[rp], jv[rp], k - NBUF, bs[rp])
                    fire(slot, iv[r], jv[r])
            return iv, jv, bs

        ng = bpw // L
        lax.fori_loop(0, ng, group, (iv0, jv0, jnp.zeros((L,), jnp.float32)))

        ivl = ii_v[pl.ds(bpw - L, L)]
        jvl = jj_v[pl.ds(bpw - L, L)]
        bsl = bi_v[pl.ds(bpw - L, L)] + bj_v[pl.ds(bpw - L, L)]
        for r in range(L - NBUF, L):
            slot = r % NBUF
            drain(slot)
            retire(slot, ivl[r], jvl[r], (ng - 1) * L + r, bsl[r])

        pltpu.sync_copy(out_v, out_hbm.at[pl.ds(base, bpw)])

    return dot


def kernel(i_indices, j_indices, wi, wj, bi, bj):
    V, D = wi.shape
    B = i_indices.shape[0]
    ii = i_indices.astype(jnp.int32)
    jj = j_indices.astype(jnp.int32)
    wit = wi.T.reshape(D // 8, 8, V)
    wjt = wj.T.reshape(D // 8, 8, V)
    return _build_dot(V, D, B)(ii, jj, wit, wjt,
                               wit[:, :, V - 128:], wjt[:, :, V - 128:],
                               bi.reshape(V), bj.reshape(V))


# speedup vs baseline: 8.9242x; 1.3581x over previous
import functools

import jax
import jax.numpy as jnp
from jax import lax
from jax.experimental import pallas as pl
from jax.experimental.pallas import tpu as pltpu
from jax.experimental.pallas import tpu_sc as plsc

L = 16
NC, NS = 2, 16
NW = NC * NS
NBUF = 8


@functools.lru_cache(maxsize=None)
def _build_dot(V, D, B):
    bpw = B // NW
    chunk = min(bpw, 128)
    nch = bpw // chunk
    vfull = V - V % 128
    nft = D // 8
    mesh = plsc.VectorSubcoreMesh(core_axis_name="c", subcore_axis_name="s")

    @functools.partial(
        pl.kernel,
        mesh=mesh,
        out_type=jax.ShapeDtypeStruct((B,), jnp.float32),
        compiler_params=pltpu.CompilerParams(
            needs_layout_passes=False, use_tc_tiling_on_sc=True),
        scratch_types=[
            pltpu.VMEM((bpw,), jnp.int32),
            pltpu.VMEM((bpw,), jnp.int32),
            pltpu.VMEM((NBUF, nft, 8, 128), jnp.float32),
            pltpu.VMEM((NBUF, nft, 8, 128), jnp.float32),
            pltpu.VMEM((nft, 8, 128), jnp.float32),
            pltpu.VMEM((nft, 8, 128), jnp.float32),
            pltpu.VMEM((bpw,), jnp.float32),
        ] + [pltpu.SemaphoreType.DMA] * (NBUF + 1),
    )
    def dot(ii_hbm, jj_hbm, wit_hbm, wjt_hbm, wtl_hbm, wtr_hbm, out_hbm,
            ii_v, jj_v, ring_i, ring_j, tail_i, tail_j,
            out_v, *sems):
        wid = lax.axis_index("s") * NC + lax.axis_index("c")
        base = wid * bpw
        tsem = sems[NBUF]
        pltpu.sync_copy(ii_hbm.at[pl.ds(base, bpw)], ii_v)
        pltpu.sync_copy(jj_hbm.at[pl.ds(base, bpw)], jj_v)

        head = [pltpu.async_copy(wtl_hbm, tail_i, tsem),
                pltpu.async_copy(wtr_hbm, tail_j, tsem)]

        lane = lax.iota(jnp.int32, L)
        ft_lo, s_lo = lane // 8, lane % 8
        ft_hi, s_hi = ft_lo + 2, s_lo
        max_tile = jnp.int32(vfull // 128 - 1)

        def fire(slot, vi, vj):
            ti = pl.multiple_of(jnp.minimum(vi >> 7, max_tile) << 7, 128)
            tj = pl.multiple_of(jnp.minimum(vj >> 7, max_tile) << 7, 128)
            pltpu.async_copy(wit_hbm.at[:, :, pl.ds(ti, 128)],
                             ring_i.at[slot], sems[slot])
            pltpu.async_copy(wjt_hbm.at[:, :, pl.ds(tj, 128)],
                             ring_j.at[slot], sems[slot])

        def drain(slot):
            pltpu.make_async_copy(
                wit_hbm.at[:, :, pl.ds(0, 128)], ring_i.at[slot],
                sems[slot]).wait()
            pltpu.make_async_copy(
                wit_hbm.at[:, :, pl.ds(0, 128)], ring_j.at[slot],
                sems[slot]).wait()

        def gather32(src, col):
            return (plsc.load_gather(src, [ft_lo, s_lo, col]),
                    plsc.load_gather(src, [ft_hi, s_hi, col]))

        def retire(slot, vi, vj, k):
            gi0, gi1 = gather32(ring_i.at[slot], jnp.broadcast_to(vi & 127, (L,)))
            gj0, gj1 = gather32(ring_j.at[slot], jnp.broadcast_to(vj & 127, (L,)))
            s = jnp.sum(gi0 * gj0 + gi1 * gj1)
            kk = jnp.broadcast_to(k, (L,))
            plsc.store_scatter(out_v, [kk], jnp.broadcast_to(s, (L,)),
                               mask=lane == 0)

            @pl.when(jnp.logical_or(vi >= vfull, vj >= vfull))
            def _():
                ci = jnp.broadcast_to(jnp.clip(vi - (V - 128), 0, 127), (L,))
                cj = jnp.broadcast_to(jnp.clip(vj - (V - 128), 0, 127), (L,))
                ti0, ti1 = gather32(tail_i, ci)
                tj0, tj1 = gather32(tail_j, cj)
                ri0, ri1 = gather32(ring_i.at[slot],
                                    jnp.broadcast_to(vi & 127, (L,)))
                rj0, rj1 = gather32(ring_j.at[slot],
                                    jnp.broadcast_to(vj & 127, (L,)))
                xi0 = jnp.where(vi >= vfull, ti0, ri0)
                xi1 = jnp.where(vi >= vfull, ti1, ri1)
                xj0 = jnp.where(vj >= vfull, tj0, rj0)
                xj1 = jnp.where(vj >= vfull, tj1, rj1)
                st = jnp.sum(xi0 * xj0 + xi1 * xj1)
                plsc.store_scatter(out_v, [kk], jnp.broadcast_to(st, (L,)),
                                   mask=lane == 0)

        iv0 = ii_v[pl.ds(0, L)]
        jv0 = jj_v[pl.ds(0, L)]
        for r in range(NBUF):
            fire(r, iv0[r], jv0[r])
        for cp in head:
            cp.wait()

        def group(g, carry):
            iv_p, jv_p = carry
            iv = ii_v[pl.ds(g * L, L)]
            jv = jj_v[pl.ds(g * L, L)]
            for r in range(L):
                slot = r % NBUF
                k = g * L + r
                if r < NBUF:
                    @pl.when(g > 0)
                    def _(slot=slot, rp=r + L - NBUF, iv_p=iv_p, jv_p=jv_p,
                          kp=k - NBUF):
                        drain(slot)
                        retire(slot, iv_p[rp], jv_p[rp], kp)
                    @pl.when(g > 0)
                    def _(slot=slot, r=r, iv=iv, jv=jv):
                        fire(slot, iv[r], jv[r])
                else:
                    drain(slot)
                    rp = r - NBUF
                    retire(slot, iv[rp], jv[rp], k - NBUF)
                    fire(slot, iv[r], jv[r])
            return iv, jv

        ng = bpw // L
        lax.fori_loop(0, ng, group, (iv0, jv0))

        ivl = ii_v[pl.ds(bpw - L, L)]
        jvl = jj_v[pl.ds(bpw - L, L)]
        for r in range(L - NBUF, L):
            slot = r % NBUF
            drain(slot)
            retire(slot, ivl[r], jvl[r], (ng - 1) * L + r)

        pltpu.sync_copy(out_v, out_hbm.at[pl.ds(base, bpw)])

    return dot


@functools.lru_cache(maxsize=None)
def _build_bias_add(V, B):
    bpw = B // NW
    chunk = min(bpw, 128)
    nch = bpw // chunk
    mesh = plsc.VectorSubcoreMesh(core_axis_name="c", subcore_axis_name="s")

    @functools.partial(
        pl.kernel,
        mesh=mesh,
        out_type=jax.ShapeDtypeStruct((B,), jnp.float32),
        compiler_params=pltpu.CompilerParams(
            needs_layout_passes=False, use_tc_tiling_on_sc=False),
        scratch_types=[
            pltpu.VMEM((nch, chunk), jnp.int32),
            pltpu.VMEM((nch, chunk), jnp.int32),
            pltpu.VMEM((bpw,), jnp.float32),
            pltpu.VMEM((bpw,), jnp.float32),
            pltpu.VMEM((bpw,), jnp.float32),
            pltpu.SemaphoreType.DMA,
        ],
    )
    def bias_add(ii_hbm, jj_hbm, bi_hbm, bj_hbm, dots_hbm, out_hbm,
                 ii_v, jj_v, bi_v, bj_v, d_v, sem):
        wid = lax.axis_index("s") * NC + lax.axis_index("c")
        base = wid * bpw
        pltpu.sync_copy(ii_hbm.at[pl.ds(wid * nch, nch)], ii_v)
        pltpu.sync_copy(jj_hbm.at[pl.ds(wid * nch, nch)], jj_v)
        pltpu.sync_copy(dots_hbm.at[pl.ds(base, bpw)], d_v)
        copies = []
        for c in range(nch):
            s = pl.ds(c * chunk, chunk)
            copies.append(pltpu.async_copy(bi_hbm.at[ii_v.at[c]], bi_v.at[s], sem))
            copies.append(pltpu.async_copy(bj_hbm.at[jj_v.at[c]], bj_v.at[s], sem))
        for cp in copies:
            cp.wait()

        def block(b0, _):
            s = pl.ds(b0 * L, L)
            d_v[s] = d_v[s] + bi_v[s] + bj_v[s]
            return 0

        lax.fori_loop(0, bpw // L, block, 0)
        pltpu.sync_copy(d_v, out_hbm.at[pl.ds(base, bpw)])

    return bias_add


def kernel(i_indices, j_indices, wi, wj, bi, bj):
    V, D = wi.shape
    B = i_indices.shape[0]
    bpw = B // NW
    chunk = min(bpw, 128)
    ii = i_indices.astype(jnp.int32)
    jj = j_indices.astype(jnp.int32)
    wit = wi.T.reshape(D // 8, 8, V)
    wjt = wj.T.reshape(D // 8, 8, V)
    dots = _build_dot(V, D, B)(ii, jj, wit, wjt,
                               wit[:, :, V - 128:], wjt[:, :, V - 128:])
    ii2 = ii.reshape(B // chunk, chunk)
    jj2 = jj.reshape(B // chunk, chunk)
    return _build_bias_add(V, B)(ii2, jj2, bi.reshape(V), bj.reshape(V), dots)
